# baseline (speedup 1.0000x reference)
import jax
import jax.numpy as jnp
from jax import lax
from jax.experimental import pallas as pl
from jax.experimental.pallas import tpu as pltpu
from jax.experimental.pallas import tpu_sc as plsc

B = 100000
NLEV = 4
D = 32
DOUT = NLEV * D
C = 640
NW = 32
NCHUNK = 5


def _sc_body(idxT, t0, t1, t2, t3, out,
             iv0, iv1, iv2, iv3, r0, r1, r2, r3, isem, gsem, wsem):
    ivs = (iv0, iv1, iv2, iv3)
    rows = (r0, r1, r2, r3)
    tables = (t0, t1, t2, t3)
    wid = lax.axis_index("s") * 2 + lax.axis_index("c")
    wh = [None] * NLEV
    for it in range(NCHUNK):
        i = wid + it * NW
        base = pl.multiple_of(jnp.minimum(i * C, B - C), 8)
        ih = [pltpu.async_copy(idxT.at[lvl, pl.ds(base, C)],
                               ivs[lvl], isem)
              for lvl in range(NLEV)]
        gh = []
        for lvl in range(NLEV):
            ih[lvl].wait()
            if wh[lvl] is not None:
                wh[lvl].wait()
            gh.append([pltpu.async_copy(
                tables[lvl].at[ivs[lvl].at[pl.ds(j * 128, 128)]],
                rows[lvl].at[pl.ds(j * 128, 128)], gsem)
                for j in range(C // 128)])
        for lvl in range(NLEV):
            for h in gh[lvl]:
                h.wait()
            wh[lvl] = pltpu.async_copy(
                rows[lvl],
                out.at[pl.ds(base, C), pl.ds(lvl * D, D)],
                wsem)
    for lvl in range(NLEV):
        wh[lvl].wait()


def kernel(code_levels, table_0, table_1, table_2, table_3):
    idxT = code_levels.T
    mesh = plsc.VectorSubcoreMesh(core_axis_name="c", subcore_axis_name="s")
    run = pl.kernel(
        _sc_body,
        out_type=jax.ShapeDtypeStruct((B, DOUT), jnp.float32),
        mesh=mesh,
        compiler_params=pltpu.CompilerParams(use_tc_tiling_on_sc=False),
        scratch_types=(
            [pltpu.VMEM((C,), jnp.int32)] * NLEV
            + [pltpu.VMEM((C, D), jnp.float32)] * NLEV
            + [pltpu.SemaphoreType.DMA] * 3
        ),
    )
    return run(idxT, table_0, table_1, table_2, table_3)


# speedup vs baseline: 1.0809x; 1.0809x over previous
import jax
import jax.numpy as jnp
from jax import lax
from jax.experimental import pallas as pl
from jax.experimental.pallas import tpu as pltpu
from jax.experimental.pallas import tpu_sc as plsc

B = 100000
NLEV = 4
D = 32
DOUT = NLEV * D
C = 512
SUB = 128
K = C // SUB
NW = 32
N_CH = -(-B // C)
MAX_IT = -(-N_CH // NW)


def _sc_body(idx0, idx1, idx2, idx3, t0, t1, t2, t3, out,
             iv0, iv1, iv2, iv3, r0, r1, r2, r3, sem):
    idxs = (idx0, idx1, idx2, idx3)
    ivs = (iv0, iv1, iv2, iv3)
    rows = (r0, r1, r2, r3)
    tables = (t0, t1, t2, t3)
    wid = lax.axis_index("s") * 2 + lax.axis_index("c")
    for it in range(MAX_IT):
        i = wid + it * NW
        @pl.when(i < N_CH)
        def _chunk():
            base = pl.multiple_of(jnp.minimum(i * C, B - C), 8)
            for lvl in range(NLEV):
                pltpu.sync_copy(idxs[lvl].at[pl.ds(base, C)], ivs[lvl])
            handles = []
            for lvl in range(NLEV):
                for j in range(K):
                    handles.append(pltpu.async_copy(
                        tables[lvl].at[ivs[lvl].at[pl.ds(j * SUB, SUB)]],
                        rows[lvl].at[pl.ds(j * SUB, SUB)],
                        sem))
            for h in handles:
                h.wait()
            for lvl in range(NLEV):
                pltpu.sync_copy(rows[lvl],
                                out.at[pl.ds(base, C), pl.ds(lvl * D, D)])


def kernel(code_levels, table_0, table_1, table_2, table_3):
    idx_cols = [code_levels[:, lvl] for lvl in range(NLEV)]
    mesh = plsc.VectorSubcoreMesh(core_axis_name="c", subcore_axis_name="s")
    run = pl.kernel(
        _sc_body,
        out_type=jax.ShapeDtypeStruct((B, DOUT), jnp.float32),
        mesh=mesh,
        compiler_params=pltpu.CompilerParams(use_tc_tiling_on_sc=False),
        scratch_types=(
            [pltpu.VMEM((C,), jnp.int32)] * NLEV
            + [pltpu.VMEM((C, D), jnp.float32)] * NLEV
            + [pltpu.SemaphoreType.DMA]
        ),
    )
    return run(*idx_cols, table_0, table_1, table_2, table_3)


# speedup vs baseline: 1.1377x; 1.0525x over previous
import jax
import jax.numpy as jnp
from jax import lax
from jax.experimental import pallas as pl
from jax.experimental.pallas import tpu as pltpu
from jax.experimental.pallas import tpu_sc as plsc

B = 100000
NLEV = 4
D = 32
DOUT = NLEV * D
C = 768
SUB = 128
K = C // SUB
NW = 32
N_CH = -(-B // C)
MAX_IT = -(-N_CH // NW)


def _sc_body(idx0, idx1, idx2, idx3, t0, t1, t2, t3, out,
             iv0, iv1, iv2, iv3, r0, r1, r2, r3, sem):
    idxs = (idx0, idx1, idx2, idx3)
    ivs = (iv0, iv1, iv2, iv3)
    rows = (r0, r1, r2, r3)
    tables = (t0, t1, t2, t3)
    wid = lax.axis_index("s") * 2 + lax.axis_index("c")
    for it in range(MAX_IT):
        i = wid + it * NW
        @pl.when(i < N_CH)
        def _chunk():
            base = pl.multiple_of(jnp.minimum(i * C, B - C), 8)
            for lvl in range(NLEV):
                pltpu.sync_copy(idxs[lvl].at[pl.ds(base, C)], ivs[lvl])
            handles = []
            for lvl in range(NLEV):
                for j in range(K):
                    handles.append(pltpu.async_copy(
                        tables[lvl].at[ivs[lvl].at[pl.ds(j * SUB, SUB)]],
                        rows[lvl].at[pl.ds(j * SUB, SUB)],
                        sem))
            for h in handles:
                h.wait()
            for lvl in range(NLEV):
                pltpu.sync_copy(rows[lvl],
                                out.at[pl.ds(base, C), pl.ds(lvl * D, D)])


def kernel(code_levels, table_0, table_1, table_2, table_3):
    idx_cols = [code_levels[:, lvl] for lvl in range(NLEV)]
    mesh = plsc.VectorSubcoreMesh(core_axis_name="c", subcore_axis_name="s")
    run = pl.kernel(
        _sc_body,
        out_type=jax.ShapeDtypeStruct((B, DOUT), jnp.float32),
        mesh=mesh,
        compiler_params=pltpu.CompilerParams(use_tc_tiling_on_sc=False),
        scratch_types=(
            [pltpu.VMEM((C,), jnp.int32)] * NLEV
            + [pltpu.VMEM((C, D), jnp.float32)] * NLEV
            + [pltpu.SemaphoreType.DMA]
        ),
    )
    return run(*idx_cols, table_0, table_1, table_2, table_3)


# speedup vs baseline: 1.1792x; 1.0365x over previous
import jax
import jax.numpy as jnp
from jax import lax
from jax.experimental import pallas as pl
from jax.experimental.pallas import tpu as pltpu
from jax.experimental.pallas import tpu_sc as plsc

B = 100000
NLEV = 4
D = 32
DOUT = NLEV * D
C = 896
SUB = 128
K = C // SUB
NW = 32
N_CH = -(-B // C)
MAX_IT = -(-N_CH // NW)


def _sc_body(idx0, idx1, idx2, idx3, t0, t1, t2, t3, out,
             iv0, iv1, iv2, iv3, r0, r1, r2, r3, sem):
    idxs = (idx0, idx1, idx2, idx3)
    ivs = (iv0, iv1, iv2, iv3)
    rows = (r0, r1, r2, r3)
    tables = (t0, t1, t2, t3)
    wid = lax.axis_index("s") * 2 + lax.axis_index("c")
    for it in range(MAX_IT):
        i = wid + it * NW
        @pl.when(i < N_CH)
        def _chunk():
            base = pl.multiple_of(jnp.minimum(i * C, B - C), 8)
            for lvl in range(NLEV):
                pltpu.sync_copy(idxs[lvl].at[pl.ds(base, C)], ivs[lvl])
            handles = []
            for lvl in range(NLEV):
                for j in range(K):
                    handles.append(pltpu.async_copy(
                        tables[lvl].at[ivs[lvl].at[pl.ds(j * SUB, SUB)]],
                        rows[lvl].at[pl.ds(j * SUB, SUB)],
                        sem))
            for h in handles:
                h.wait()
            for lvl in range(NLEV):
                pltpu.sync_copy(rows[lvl],
                                out.at[pl.ds(base, C), pl.ds(lvl * D, D)])


def kernel(code_levels, table_0, table_1, table_2, table_3):
    idx_cols = [code_levels[:, lvl] for lvl in range(NLEV)]
    mesh = plsc.VectorSubcoreMesh(core_axis_name="c", subcore_axis_name="s")
    run = pl.kernel(
        _sc_body,
        out_type=jax.ShapeDtypeStruct((B, DOUT), jnp.float32),
        mesh=mesh,
        compiler_params=pltpu.CompilerParams(use_tc_tiling_on_sc=False),
        scratch_types=(
            [pltpu.VMEM((C,), jnp.int32)] * NLEV
            + [pltpu.VMEM((C, D), jnp.float32)] * NLEV
            + [pltpu.SemaphoreType.DMA]
        ),
    )
    return run(*idx_cols, table_0, table_1, table_2, table_3)
